# baseline (speedup 1.0000x reference)
import functools

import jax
import jax.numpy as jnp
from jax import lax
from jax.experimental import pallas as pl
from jax.experimental.pallas import tpu as pltpu
from jax.experimental.pallas import tpu_sc as plsc

N_NODES = 10000
N_EDGES = 320000
D = 128
N_GRAPHS = 64

NC = 2
NS = 16
NW = NC * NS
CHUNK = 128

F32 = jnp.float32


def _span_copy(src, dst, s, nbins):
    rpt = (nbins // NS) & ~7
    if rpt == 0:
        @pl.when(s == 0)
        def _():
            pltpu.sync_copy(src, dst)
        return
    tail = nbins - NS * rpt
    base = pl.multiple_of(s * rpt, 8)
    pltpu.sync_copy(src.at[pl.ds(base, rpt)], dst.at[pl.ds(base, rpt)])
    if tail:
        @pl.when(s == NS - 1)
        def _():
            pltpu.sync_copy(src.at[pl.ds(NS * rpt, tail)],
                            dst.at[pl.ds(NS * rpt, tail)])

def _sc_hist(dst_rs, ones_e0, zeros, nbins, nalloc, k):
    mesh = plsc.VectorSubcoreMesh(core_axis_name="c", subcore_axis_name="s",
                                  num_cores=NC, num_subcores=NS)
    HBUF = 4

    @functools.partial(
        pl.kernel,
        out_type=jax.ShapeDtypeStruct((NC, nbins, D), F32),
        mesh=mesh,
        scratch_types=[
            pltpu.VMEM((k, CHUNK), jnp.int32),
            pltpu.VMEM((CHUNK, D), F32),
            pltpu.SemaphoreType.DMA((HBUF,)),
            pltpu.VMEM_SHARED((nalloc, D), F32),
        ],
    )
    def hist(dst_hbm, ones_hbm, zeros_hbm, out_hbm, idx_v, ones_v, semh, acc):
        c = lax.axis_index("c")
        s = lax.axis_index("s")
        wid = c * NS + s
        pltpu.sync_copy(dst_hbm.at[wid], idx_v)
        pltpu.sync_copy(ones_hbm, ones_v)
        _span_copy(zeros_hbm, acc, s, nbins)
        plsc.subcore_barrier()

        def wait_h(j, b):
            pltpu.make_async_copy(ones_v, acc.at[idx_v.at[j]],
                                  semh.at[b]).wait()

        def body(g, carry):
            for b in range(HBUF):
                pltpu.async_copy(ones_v, acc.at[idx_v.at[g * HBUF + b]],
                                 semh.at[b], add=True)
            for b in range(HBUF):
                wait_h(g * HBUF + b, b)
            return carry
        if k // HBUF:
            lax.fori_loop(0, k // HBUF, body, 0)
        for j in range(k - k % HBUF, k):
            pltpu.sync_copy(ones_v, acc.at[idx_v.at[j]], add=True)
        plsc.subcore_barrier()
        _span_copy(acc, out_hbm.at[c], s, nbins)

    return hist(dst_rs, ones_e0, zeros)


def _sc_agg(table, src_rs, dst_rs, zeros, nbins, nalloc, k):
    mesh = plsc.VectorSubcoreMesh(core_axis_name="c", subcore_axis_name="s",
                                  num_cores=NC, num_subcores=NS)

    NBUF = 2
    PH = 40
    phases = [(p0, min(PH, k - p0)) for p0 in range(0, k, PH)]
    kmax = max(kp for _, kp in phases)
    assert min(kp for _, kp in phases) >= NBUF

    @functools.partial(
        pl.kernel,
        out_type=jax.ShapeDtypeStruct((NC, nbins, D), F32),
        mesh=mesh,
        scratch_types=[
            pltpu.VMEM((kmax, CHUNK), jnp.int32),
            pltpu.VMEM((kmax, CHUNK), jnp.int32),
            pltpu.VMEM((NBUF, CHUNK, D), F32),
            pltpu.SemaphoreType.DMA((NBUF,)),
            pltpu.SemaphoreType.DMA((NBUF,)),
            pltpu.VMEM_SHARED((nalloc, D), F32),
        ],
    )
    def agg(table_hbm, src_hbm, dst_hbm, zeros_hbm, out_hbm,
            src_v, dst_v, rows_v, semg, sems, acc):
        c = lax.axis_index("c")
        s = lax.axis_index("s")
        wid = c * NS + s

        def wait_g(j, b):
            pltpu.make_async_copy(table_hbm.at[src_v.at[j]],
                                  rows_v.at[b], semg.at[b]).wait()

        def wait_s(j, b):
            pltpu.make_async_copy(rows_v.at[b],
                                  acc.at[dst_v.at[j]], sems.at[b]).wait()

        first = True
        for p0, kp in phases:
            pltpu.sync_copy(src_hbm.at[wid, pl.ds(p0, kp)],
                            src_v.at[pl.ds(0, kp)])
            pltpu.sync_copy(dst_hbm.at[wid, pl.ds(p0, kp)],
                            dst_v.at[pl.ds(0, kp)])
            for b in range(NBUF):
                pltpu.async_copy(table_hbm.at[src_v.at[b]], rows_v.at[b],
                                 semg.at[b])
            if first:
                _span_copy(zeros_hbm, acc, s, nbins)
                plsc.subcore_barrier()
                first = False

            n_main = (kp - NBUF) // NBUF
            base = n_main * NBUF

            def body(g, carry):
                for b in range(NBUF):
                    j = g * NBUF + b
                    wait_g(j, b)
                    pltpu.async_copy(rows_v.at[b], acc.at[dst_v.at[j]],
                                     sems.at[b], add=True)
                    wait_s(j, b)
                    pltpu.async_copy(table_hbm.at[src_v.at[j + NBUF]],
                                     rows_v.at[b], semg.at[b])
                return carry
            if n_main:
                lax.fori_loop(0, n_main, body, 0)

            for b in range(NBUF):
                wait_g(base + b, b)
                pltpu.async_copy(rows_v.at[b], acc.at[dst_v.at[base + b]],
                                 sems.at[b], add=True)
            for j in range(base + NBUF, kp):
                b = j % NBUF
                wait_s(j - NBUF, b)
                pltpu.sync_copy(table_hbm.at[src_v.at[j]], rows_v.at[b])
                pltpu.async_copy(rows_v.at[b], acc.at[dst_v.at[j]],
                                 sems.at[b], add=True)
            for b in range(NBUF):
                wait_s(kp - 1, b)
        plsc.subcore_barrier()
        _span_copy(acc, out_hbm.at[c], s, nbins)

    return agg(table, src_rs, dst_rs, zeros)


R = 400
GRID = N_NODES // R


def _tc_prep(x, W1, degp):
    def body(x_ref, w_ref, degp_ref, h_ref, hs_ref, dis_ref, inv_ref):
        p = degp_ref[...]
        d = p[0, :, 0:1] + p[1, :, 0:1] + 1.0
        dis = lax.rsqrt(d)
        inv = 1.0 / d
        h = jnp.dot(x_ref[...], w_ref[...], preferred_element_type=F32)
        h_ref[...] = h
        hs_ref[...] = h * dis
        dis_ref[...] = dis
        inv_ref[...] = inv

    return pl.pallas_call(
        body,
        grid=(GRID,),
        in_specs=[
            pl.BlockSpec((R, D), lambda i: (i, 0)),
            pl.BlockSpec((D, D), lambda i: (0, 0)),
            pl.BlockSpec((NC, R, D), lambda i: (0, i, 0)),
        ],
        out_specs=[
            pl.BlockSpec((R, D), lambda i: (i, 0)),
            pl.BlockSpec((R, D), lambda i: (i, 0)),
            pl.BlockSpec((R, 1), lambda i: (i, 0)),
            pl.BlockSpec((R, 1), lambda i: (i, 0)),
        ],
        out_shape=[
            jax.ShapeDtypeStruct((N_NODES, D), F32),
            jax.ShapeDtypeStruct((N_NODES, D), F32),
            jax.ShapeDtypeStruct((N_NODES, 1), F32),
            jax.ShapeDtypeStruct((N_NODES, 1), F32),
        ],
    )(x, W1, degp)


def _tc_combine(P, H, dis, inv, b, Wn):
    def body(p_ref, h_ref, dis_ref, inv_ref, b_ref, w_ref, hn_ref, hsn_ref):
        p = p_ref[...]
        dis = dis_ref[...]
        xv = jnp.maximum(dis * (p[0] + p[1]) + inv_ref[...] * h_ref[...]
                         + b_ref[...], 0.0)
        hn = jnp.dot(xv, w_ref[...], preferred_element_type=F32)
        hn_ref[...] = hn
        hsn_ref[...] = hn * dis

    return pl.pallas_call(
        body,
        grid=(GRID,),
        in_specs=[
            pl.BlockSpec((NC, R, D), lambda i: (0, i, 0)),
            pl.BlockSpec((R, D), lambda i: (i, 0)),
            pl.BlockSpec((R, 1), lambda i: (i, 0)),
            pl.BlockSpec((R, 1), lambda i: (i, 0)),
            pl.BlockSpec((1, D), lambda i: (0, 0)),
            pl.BlockSpec((D, D), lambda i: (0, 0)),
        ],
        out_specs=[
            pl.BlockSpec((R, D), lambda i: (i, 0)),
            pl.BlockSpec((R, D), lambda i: (i, 0)),
        ],
        out_shape=[
            jax.ShapeDtypeStruct((N_NODES, D), F32),
            jax.ShapeDtypeStruct((N_NODES, D), F32),
        ],
    )(P, H, dis, inv, b, Wn)


def _tc_combine_last(P, H, dis, inv, b):
    def body(p_ref, h_ref, dis_ref, inv_ref, b_ref, x_ref):
        p = p_ref[...]
        x_ref[...] = jnp.maximum(dis_ref[...] * (p[0] + p[1])
                                 + inv_ref[...] * h_ref[...] + b_ref[...], 0.0)

    return pl.pallas_call(
        body,
        grid=(GRID,),
        in_specs=[
            pl.BlockSpec((NC, R, D), lambda i: (0, i, 0)),
            pl.BlockSpec((R, D), lambda i: (i, 0)),
            pl.BlockSpec((R, 1), lambda i: (i, 0)),
            pl.BlockSpec((R, 1), lambda i: (i, 0)),
            pl.BlockSpec((1, D), lambda i: (0, 0)),
        ],
        out_specs=pl.BlockSpec((R, D), lambda i: (i, 0)),
        out_shape=jax.ShapeDtypeStruct((N_NODES, D), F32),
    )(P, H, dis, inv, b)


def _tc_final(Pp, cntp):
    def body(p_ref, c_ref, o_ref):
        p = p_ref[...]
        sums = p[0, :N_GRAPHS, :] + p[1, :N_GRAPHS, :]
        cn = c_ref[...]
        cnt = cn[0, :N_GRAPHS, 0:1] + cn[1, :N_GRAPHS, 0:1]
        o_ref[...] = sums / jnp.maximum(cnt, 1.0)

    return pl.pallas_call(
        body,
        in_specs=[
            pl.BlockSpec((NC, N_GRAPHS + 1, D), lambda: (0, 0, 0)),
            pl.BlockSpec((NC, N_GRAPHS + 1, D), lambda: (0, 0, 0)),
        ],
        out_specs=pl.BlockSpec((N_GRAPHS, D), lambda: (0, 0)),
        out_shape=jax.ShapeDtypeStruct((N_GRAPHS, D), F32),
    )(Pp, cntp)


def kernel(x, edge_index, batch, W1, b1, W2, b2, W3, b3):
    k_e = -(-N_EDGES // (NW * CHUNK))
    e_pad = k_e * NW * CHUNK - N_EDGES
    src = jnp.concatenate(
        [edge_index[0].astype(jnp.int32),
         jnp.zeros((e_pad,), jnp.int32)]).reshape(NW, k_e, CHUNK)
    dst = jnp.concatenate(
        [edge_index[1].astype(jnp.int32),
         jnp.full((e_pad,), N_NODES, jnp.int32)]).reshape(NW, k_e, CHUNK)
    bat = batch.astype(jnp.int32)
    NALLOC = N_NODES + 8

    k_p = -(-N_NODES // (NW * CHUNK))
    n_pad = k_p * NW * CHUNK
    pool_src = jnp.concatenate(
        [jnp.arange(N_NODES, dtype=jnp.int32),
         jnp.zeros((n_pad - N_NODES,), jnp.int32)]).reshape(NW, k_p, CHUNK)
    pool_dst = jnp.concatenate(
        [bat, jnp.full((n_pad - N_NODES,), N_GRAPHS, jnp.int32)]
    ).reshape(NW, k_p, CHUNK)

    ones_e0 = jnp.zeros((CHUNK, D), F32).at[:, 0].set(1.0)
    z_nodes = jnp.zeros((N_NODES, D), F32)
    z_pool = jnp.zeros((N_GRAPHS + 1, D), F32)

    degp = _sc_hist(dst, ones_e0, z_nodes, N_NODES, NALLOC, k_e)
    cntp = _sc_hist(pool_dst, ones_e0, z_pool, N_GRAPHS + 1, N_GRAPHS + 1, k_p)

    b1r, b2r, b3r = (b.reshape(1, D) for b in (b1, b2, b3))

    H1, Hs1, dis, inv = _tc_prep(x, W1, degp)
    P1 = _sc_agg(Hs1, src, dst, z_nodes, N_NODES, NALLOC, k_e)
    H2, Hs2 = _tc_combine(P1, H1, dis, inv, b1r, W2)
    P2 = _sc_agg(Hs2, src, dst, z_nodes, N_NODES, NALLOC, k_e)
    H3, Hs3 = _tc_combine(P2, H2, dis, inv, b2r, W3)
    P3 = _sc_agg(Hs3, src, dst, z_nodes, N_NODES, NALLOC, k_e)
    X3 = _tc_combine_last(P3, H3, dis, inv, b3r)
    Pp = _sc_agg(X3, pool_src, pool_dst, z_pool, N_GRAPHS + 1,
                 N_GRAPHS + 1, k_p)
    return _tc_final(Pp, cntp)


# speedup vs baseline: 1.0008x; 1.0008x over previous
import functools

import jax
import jax.numpy as jnp
from jax import lax
from jax.experimental import pallas as pl
from jax.experimental.pallas import tpu as pltpu
from jax.experimental.pallas import tpu_sc as plsc

N_NODES = 10000
N_EDGES = 320000
D = 128
N_GRAPHS = 64

NC = 2
NS = 16
NW = NC * NS
CHUNK = 128

F32 = jnp.float32


def _span_copy(src, dst, s, nbins):
    rpt = (nbins // NS) & ~7
    if rpt == 0:
        @pl.when(s == 0)
        def _():
            pltpu.sync_copy(src.at[pl.ds(0, nbins)], dst.at[pl.ds(0, nbins)])
        return
    tail = nbins - NS * rpt
    base = pl.multiple_of(s * rpt, 8)
    pltpu.sync_copy(src.at[pl.ds(base, rpt)], dst.at[pl.ds(base, rpt)])
    if tail:
        @pl.when(s == NS - 1)
        def _():
            pltpu.sync_copy(src.at[pl.ds(NS * rpt, tail)],
                            dst.at[pl.ds(NS * rpt, tail)])

def _sc_hist(dst_rs, ones_e0, zeros, nbins, nalloc, k):
    mesh = plsc.VectorSubcoreMesh(core_axis_name="c", subcore_axis_name="s",
                                  num_cores=NC, num_subcores=NS)
    HBUF = 4

    @functools.partial(
        pl.kernel,
        out_type=jax.ShapeDtypeStruct((NC, nbins, D), F32),
        mesh=mesh,
        scratch_types=[
            pltpu.VMEM((k, CHUNK), jnp.int32),
            pltpu.VMEM((CHUNK, D), F32),
            pltpu.SemaphoreType.DMA((HBUF,)),
            pltpu.VMEM_SHARED((nalloc, D), F32),
        ],
    )
    def hist(dst_hbm, ones_hbm, zeros_hbm, out_hbm, idx_v, ones_v, semh, acc):
        c = lax.axis_index("c")
        s = lax.axis_index("s")
        wid = c * NS + s
        pltpu.sync_copy(dst_hbm.at[wid], idx_v)
        pltpu.sync_copy(ones_hbm, ones_v)
        _span_copy(zeros_hbm, acc, s, nbins)
        plsc.subcore_barrier()

        def wait_h(j, b):
            pltpu.make_async_copy(ones_v, acc.at[idx_v.at[j]],
                                  semh.at[b]).wait()

        def body(g, carry):
            for b in range(HBUF):
                pltpu.async_copy(ones_v, acc.at[idx_v.at[g * HBUF + b]],
                                 semh.at[b], add=True)
            for b in range(HBUF):
                wait_h(g * HBUF + b, b)
            return carry
        if k // HBUF:
            lax.fori_loop(0, k // HBUF, body, 0)
        for j in range(k - k % HBUF, k):
            pltpu.sync_copy(ones_v, acc.at[idx_v.at[j]], add=True)
        plsc.subcore_barrier()
        _span_copy(acc, out_hbm.at[c], s, nbins)

    return hist(dst_rs, ones_e0, zeros)


def _sc_agg(table, src_rs, dst_rs, zeros, nbins, nalloc, k):
    mesh = plsc.VectorSubcoreMesh(core_axis_name="c", subcore_axis_name="s",
                                  num_cores=NC, num_subcores=NS)

    NBUF = 2
    PH = 40
    phases = [(p0, min(PH, k - p0)) for p0 in range(0, k, PH)]
    kmax = max(kp for _, kp in phases)
    assert min(kp for _, kp in phases) >= NBUF

    @functools.partial(
        pl.kernel,
        out_type=jax.ShapeDtypeStruct((NC, nbins, D), F32),
        mesh=mesh,
        scratch_types=[
            pltpu.VMEM((kmax, CHUNK), jnp.int32),
            pltpu.VMEM((kmax, CHUNK), jnp.int32),
            pltpu.VMEM((NBUF, CHUNK, D), F32),
            pltpu.SemaphoreType.DMA((NBUF,)),
            pltpu.SemaphoreType.DMA((NBUF,)),
            pltpu.VMEM_SHARED((nalloc, D), F32),
        ],
    )
    def agg(table_hbm, src_hbm, dst_hbm, zeros_hbm, out_hbm,
            src_v, dst_v, rows_v, semg, sems, acc):
        c = lax.axis_index("c")
        s = lax.axis_index("s")
        wid = c * NS + s

        def wait_g(j, b):
            pltpu.make_async_copy(table_hbm.at[src_v.at[j]],
                                  rows_v.at[b], semg.at[b]).wait()

        def wait_s(j, b):
            pltpu.make_async_copy(rows_v.at[b],
                                  acc.at[dst_v.at[j]], sems.at[b]).wait()

        first = True
        for p0, kp in phases:
            pltpu.sync_copy(src_hbm.at[wid, pl.ds(p0, kp)],
                            src_v.at[pl.ds(0, kp)])
            pltpu.sync_copy(dst_hbm.at[wid, pl.ds(p0, kp)],
                            dst_v.at[pl.ds(0, kp)])
            for b in range(NBUF):
                pltpu.async_copy(table_hbm.at[src_v.at[b]], rows_v.at[b],
                                 semg.at[b])
            if first:
                _span_copy(zeros_hbm, acc, s, nbins)
                plsc.subcore_barrier()
                first = False

            n_main = (kp - NBUF) // NBUF
            base = n_main * NBUF

            def body(g, carry):
                for b in range(NBUF):
                    j = g * NBUF + b
                    wait_g(j, b)
                    pltpu.async_copy(rows_v.at[b], acc.at[dst_v.at[j]],
                                     sems.at[b], add=True)
                    wait_s(j, b)
                    pltpu.async_copy(table_hbm.at[src_v.at[j + NBUF]],
                                     rows_v.at[b], semg.at[b])
                return carry
            if n_main:
                lax.fori_loop(0, n_main, body, 0)

            for b in range(NBUF):
                wait_g(base + b, b)
                pltpu.async_copy(rows_v.at[b], acc.at[dst_v.at[base + b]],
                                 sems.at[b], add=True)
            for j in range(base + NBUF, kp):
                b = j % NBUF
                wait_s(j - NBUF, b)
                pltpu.sync_copy(table_hbm.at[src_v.at[j]], rows_v.at[b])
                pltpu.async_copy(rows_v.at[b], acc.at[dst_v.at[j]],
                                 sems.at[b], add=True)
            for b in range(NBUF):
                wait_s(kp - 1, b)
        plsc.subcore_barrier()
        _span_copy(acc, out_hbm.at[c], s, nbins)

    return agg(table, src_rs, dst_rs, zeros)


R = 400
GRID = N_NODES // R


def _tc_prep(x, W1, degp):
    def body(x_ref, w_ref, degp_ref, h_ref, hs_ref, dis_ref, inv_ref):
        p = degp_ref[...]
        d = p[0, :, 0:1] + p[1, :, 0:1] + 1.0
        dis = lax.rsqrt(d)
        inv = 1.0 / d
        h = jnp.dot(x_ref[...], w_ref[...], preferred_element_type=F32)
        h_ref[...] = h
        hs_ref[...] = h * dis
        dis_ref[...] = dis
        inv_ref[...] = inv

    return pl.pallas_call(
        body,
        grid=(GRID,),
        in_specs=[
            pl.BlockSpec((R, D), lambda i: (i, 0)),
            pl.BlockSpec((D, D), lambda i: (0, 0)),
            pl.BlockSpec((NC, R, D), lambda i: (0, i, 0)),
        ],
        out_specs=[
            pl.BlockSpec((R, D), lambda i: (i, 0)),
            pl.BlockSpec((R, D), lambda i: (i, 0)),
            pl.BlockSpec((R, 1), lambda i: (i, 0)),
            pl.BlockSpec((R, 1), lambda i: (i, 0)),
        ],
        out_shape=[
            jax.ShapeDtypeStruct((N_NODES, D), F32),
            jax.ShapeDtypeStruct((N_NODES, D), F32),
            jax.ShapeDtypeStruct((N_NODES, 1), F32),
            jax.ShapeDtypeStruct((N_NODES, 1), F32),
        ],
    )(x, W1, degp)


def _tc_combine(P, H, dis, inv, b, Wn):
    def body(p_ref, h_ref, dis_ref, inv_ref, b_ref, w_ref, hn_ref, hsn_ref):
        p = p_ref[...]
        dis = dis_ref[...]
        xv = jnp.maximum(dis * (p[0] + p[1]) + inv_ref[...] * h_ref[...]
                         + b_ref[...], 0.0)
        hn = jnp.dot(xv, w_ref[...], preferred_element_type=F32)
        hn_ref[...] = hn
        hsn_ref[...] = hn * dis

    return pl.pallas_call(
        body,
        grid=(GRID,),
        in_specs=[
            pl.BlockSpec((NC, R, D), lambda i: (0, i, 0)),
            pl.BlockSpec((R, D), lambda i: (i, 0)),
            pl.BlockSpec((R, 1), lambda i: (i, 0)),
            pl.BlockSpec((R, 1), lambda i: (i, 0)),
            pl.BlockSpec((1, D), lambda i: (0, 0)),
            pl.BlockSpec((D, D), lambda i: (0, 0)),
        ],
        out_specs=[
            pl.BlockSpec((R, D), lambda i: (i, 0)),
            pl.BlockSpec((R, D), lambda i: (i, 0)),
        ],
        out_shape=[
            jax.ShapeDtypeStruct((N_NODES, D), F32),
            jax.ShapeDtypeStruct((N_NODES, D), F32),
        ],
    )(P, H, dis, inv, b, Wn)


def _tc_combine_last(P, H, dis, inv, b):
    def body(p_ref, h_ref, dis_ref, inv_ref, b_ref, x_ref):
        p = p_ref[...]
        x_ref[...] = jnp.maximum(dis_ref[...] * (p[0] + p[1])
                                 + inv_ref[...] * h_ref[...] + b_ref[...], 0.0)

    return pl.pallas_call(
        body,
        grid=(GRID,),
        in_specs=[
            pl.BlockSpec((NC, R, D), lambda i: (0, i, 0)),
            pl.BlockSpec((R, D), lambda i: (i, 0)),
            pl.BlockSpec((R, 1), lambda i: (i, 0)),
            pl.BlockSpec((R, 1), lambda i: (i, 0)),
            pl.BlockSpec((1, D), lambda i: (0, 0)),
        ],
        out_specs=pl.BlockSpec((R, D), lambda i: (i, 0)),
        out_shape=jax.ShapeDtypeStruct((N_NODES, D), F32),
    )(P, H, dis, inv, b)


def _tc_final(Pp, cntp):
    def body(p_ref, c_ref, o_ref):
        p = p_ref[...]
        sums = p[0, :N_GRAPHS, :] + p[1, :N_GRAPHS, :]
        cn = c_ref[...]
        cnt = cn[0, :N_GRAPHS, 0:1] + cn[1, :N_GRAPHS, 0:1]
        o_ref[...] = sums / jnp.maximum(cnt, 1.0)

    return pl.pallas_call(
        body,
        in_specs=[
            pl.BlockSpec((NC, N_GRAPHS + 1, D), lambda: (0, 0, 0)),
            pl.BlockSpec((NC, N_GRAPHS + 1, D), lambda: (0, 0, 0)),
        ],
        out_specs=pl.BlockSpec((N_GRAPHS, D), lambda: (0, 0)),
        out_shape=jax.ShapeDtypeStruct((N_GRAPHS, D), F32),
    )(Pp, cntp)


def kernel(x, edge_index, batch, W1, b1, W2, b2, W3, b3):
    k_e = -(-N_EDGES // (NW * CHUNK))
    e_pad = k_e * NW * CHUNK - N_EDGES
    src = jnp.concatenate(
        [edge_index[0].astype(jnp.int32),
         jnp.zeros((e_pad,), jnp.int32)]).reshape(NW, k_e, CHUNK)
    dst = jnp.concatenate(
        [edge_index[1].astype(jnp.int32),
         N_NODES + jnp.arange(e_pad, dtype=jnp.int32) % 128]
    ).reshape(NW, k_e, CHUNK)
    bat = batch.astype(jnp.int32)
    NALLOC = N_NODES + 128
    NALLOC_P = 128

    k_p = -(-N_NODES // (NW * CHUNK))
    n_pad = k_p * NW * CHUNK
    pool_src = jnp.concatenate(
        [jnp.arange(N_NODES, dtype=jnp.int32),
         jnp.zeros((n_pad - N_NODES,), jnp.int32)]).reshape(NW, k_p, CHUNK)
    pool_dst = jnp.concatenate(
        [bat, N_GRAPHS + jnp.arange(n_pad - N_NODES, dtype=jnp.int32)
         % (NALLOC_P - N_GRAPHS)]).reshape(NW, k_p, CHUNK)

    ones_e0 = jnp.zeros((CHUNK, D), F32).at[:, 0].set(1.0)
    z_nodes = jnp.zeros((N_NODES, D), F32)
    z_pool = jnp.zeros((N_GRAPHS + 1, D), F32)

    degp = _sc_hist(dst, ones_e0, z_nodes, N_NODES, NALLOC, k_e)
    cntp = _sc_hist(pool_dst, ones_e0, z_pool, N_GRAPHS + 1, NALLOC_P, k_p)

    b1r, b2r, b3r = (b.reshape(1, D) for b in (b1, b2, b3))

    H1, Hs1, dis, inv = _tc_prep(x, W1, degp)
    P1 = _sc_agg(Hs1, src, dst, z_nodes, N_NODES, NALLOC, k_e)
    H2, Hs2 = _tc_combine(P1, H1, dis, inv, b1r, W2)
    P2 = _sc_agg(Hs2, src, dst, z_nodes, N_NODES, NALLOC, k_e)
    H3, Hs3 = _tc_combine(P2, H2, dis, inv, b2r, W3)
    P3 = _sc_agg(Hs3, src, dst, z_nodes, N_NODES, NALLOC, k_e)
    X3 = _tc_combine_last(P3, H3, dis, inv, b3r)
    Pp = _sc_agg(X3, pool_src, pool_dst, z_pool, N_GRAPHS + 1,
                 NALLOC_P, k_p)
    return _tc_final(Pp, cntp)


# speedup vs baseline: 1.8430x; 1.8415x over previous
import functools

import jax
import jax.numpy as jnp
from jax import lax
from jax.experimental import pallas as pl
from jax.experimental.pallas import tpu as pltpu
from jax.experimental.pallas import tpu_sc as plsc

N_NODES = 10000
N_EDGES = 320000
D = 128
N_GRAPHS = 64

NC = 2
NS = 16
NW = NC * NS
CHUNK = 128

F32 = jnp.float32


def _span_copy(src, dst, s, nbins):
    rpt = (nbins // NS) & ~7
    if rpt == 0:
        @pl.when(s == 0)
        def _():
            pltpu.sync_copy(src.at[pl.ds(0, nbins)], dst.at[pl.ds(0, nbins)])
        return
    tail = nbins - NS * rpt
    base = pl.multiple_of(s * rpt, 8)
    pltpu.sync_copy(src.at[pl.ds(base, rpt)], dst.at[pl.ds(base, rpt)])
    if tail:
        @pl.when(s == NS - 1)
        def _():
            pltpu.sync_copy(src.at[pl.ds(NS * rpt, tail)],
                            dst.at[pl.ds(NS * rpt, tail)])

def _sc_hist(dst_rs, ones_e0, zeros, nbins, nalloc, k):
    mesh = plsc.VectorSubcoreMesh(core_axis_name="c", subcore_axis_name="s",
                                  num_cores=NC, num_subcores=NS)
    HBUF = 4

    @functools.partial(
        pl.kernel,
        out_type=jax.ShapeDtypeStruct((NC, nbins, D), F32),
        mesh=mesh,
        scratch_types=[
            pltpu.VMEM((k, CHUNK), jnp.int32),
            pltpu.VMEM((CHUNK, D), F32),
            pltpu.SemaphoreType.DMA((HBUF,)),
            pltpu.VMEM_SHARED((nalloc, D), F32),
        ],
    )
    def hist(dst_hbm, ones_hbm, zeros_hbm, out_hbm, idx_v, ones_v, semh, acc):
        c = lax.axis_index("c")
        s = lax.axis_index("s")
        wid = c * NS + s
        pltpu.sync_copy(dst_hbm.at[wid], idx_v)
        pltpu.sync_copy(ones_hbm, ones_v)
        _span_copy(zeros_hbm, acc, s, nbins)
        plsc.subcore_barrier()

        def wait_h(j, b):
            pltpu.make_async_copy(ones_v, acc.at[idx_v.at[j]],
                                  semh.at[b]).wait()

        def body(g, carry):
            for b in range(HBUF):
                pltpu.async_copy(ones_v, acc.at[idx_v.at[g * HBUF + b]],
                                 semh.at[b], add=True)
            for b in range(HBUF):
                wait_h(g * HBUF + b, b)
            return carry
        if k // HBUF:
            lax.fori_loop(0, k // HBUF, body, 0)
        for j in range(k - k % HBUF, k):
            pltpu.sync_copy(ones_v, acc.at[idx_v.at[j]], add=True)
        plsc.subcore_barrier()
        _span_copy(acc, out_hbm.at[c], s, nbins)

    return hist(dst_rs, ones_e0, zeros)


def _sc_agg(table, src_rs, dst_rs, zeros, nbins, nalloc, k):
    mesh = plsc.VectorSubcoreMesh(core_axis_name="c", subcore_axis_name="s",
                                  num_cores=NC, num_subcores=NS)

    NBUF = 2
    PH = 40
    phases = [(p0, min(PH, k - p0)) for p0 in range(0, k, PH)]
    kmax = max(kp for _, kp in phases)
    assert min(kp for _, kp in phases) >= NBUF

    @functools.partial(
        pl.kernel,
        out_type=jax.ShapeDtypeStruct((NC, nbins, D), F32),
        mesh=mesh,
        scratch_types=[
            pltpu.VMEM((kmax, CHUNK), jnp.int32),
            pltpu.VMEM((kmax, CHUNK), jnp.int32),
            pltpu.VMEM((NBUF, CHUNK, D), F32),
            pltpu.SemaphoreType.DMA((NBUF,)),
            pltpu.SemaphoreType.DMA((NBUF,)),
            pltpu.VMEM_SHARED((nalloc, D), F32),
        ],
    )
    def agg(table_hbm, src_hbm, dst_hbm, zeros_hbm, out_hbm,
            src_v, dst_v, rows_v, semg, sems, acc):
        c = lax.axis_index("c")
        s = lax.axis_index("s")
        wid = c * NS + s

        def wait_g(j, b):
            pltpu.make_async_copy(table_hbm.at[src_v.at[j]],
                                  rows_v.at[b], semg.at[b]).wait()

        def wait_s(j, b):
            pltpu.make_async_copy(rows_v.at[b],
                                  acc.at[dst_v.at[j]], sems.at[b]).wait()

        first = True
        for p0, kp in phases:
            pltpu.sync_copy(src_hbm.at[wid, pl.ds(p0, kp)],
                            src_v.at[pl.ds(0, kp)])
            pltpu.sync_copy(dst_hbm.at[wid, pl.ds(p0, kp)],
                            dst_v.at[pl.ds(0, kp)])
            for b in range(NBUF):
                pltpu.async_copy(table_hbm.at[src_v.at[b]], rows_v.at[b],
                                 semg.at[b])
            if first:
                _span_copy(zeros_hbm, acc, s, nbins)
                plsc.subcore_barrier()
                first = False

            n_main = (kp - NBUF) // NBUF
            base = n_main * NBUF

            def body(g, carry):
                for b in range(NBUF):
                    j = g * NBUF + b
                    wait_g(j, b)
                    pltpu.async_copy(rows_v.at[b], acc.at[dst_v.at[j]],
                                     sems.at[b], add=True)
                    wait_s(j, b)
                    pltpu.async_copy(table_hbm.at[src_v.at[j + NBUF]],
                                     rows_v.at[b], semg.at[b])
                return carry
            if n_main:
                lax.fori_loop(0, n_main, body, 0)

            for b in range(NBUF):
                wait_g(base + b, b)
                pltpu.async_copy(rows_v.at[b], acc.at[dst_v.at[base + b]],
                                 sems.at[b], add=True)
            for j in range(base + NBUF, kp):
                b = j % NBUF
                wait_s(j - NBUF, b)
                pltpu.sync_copy(table_hbm.at[src_v.at[j]], rows_v.at[b])
                pltpu.async_copy(rows_v.at[b], acc.at[dst_v.at[j]],
                                 sems.at[b], add=True)
            for b in range(NBUF):
                wait_s(kp - 1, b)
        plsc.subcore_barrier()
        _span_copy(acc, out_hbm.at[c], s, nbins)

    return agg(table, src_rs, dst_rs, zeros)


R = 400
GRID = N_NODES // R


def _tc_prep(x, W1, degp):
    def body(x_ref, w_ref, degp_ref, h_ref, hs_ref, dis_ref, inv_ref):
        p = degp_ref[...]
        d = p[0, :, 0:1] + p[1, :, 0:1] + 1.0
        dis = lax.rsqrt(d)
        inv = 1.0 / d
        h = jnp.dot(x_ref[...], w_ref[...], preferred_element_type=F32)
        h_ref[...] = h
        hs_ref[...] = h * dis
        dis_ref[...] = dis
        inv_ref[...] = inv

    return pl.pallas_call(
        body,
        grid=(GRID,),
        in_specs=[
            pl.BlockSpec((R, D), lambda i: (i, 0)),
            pl.BlockSpec((D, D), lambda i: (0, 0)),
            pl.BlockSpec((NC, R, D), lambda i: (0, i, 0)),
        ],
        out_specs=[
            pl.BlockSpec((R, D), lambda i: (i, 0)),
            pl.BlockSpec((R, D), lambda i: (i, 0)),
            pl.BlockSpec((R, 1), lambda i: (i, 0)),
            pl.BlockSpec((R, 1), lambda i: (i, 0)),
        ],
        out_shape=[
            jax.ShapeDtypeStruct((N_NODES, D), F32),
            jax.ShapeDtypeStruct((N_NODES, D), F32),
            jax.ShapeDtypeStruct((N_NODES, 1), F32),
            jax.ShapeDtypeStruct((N_NODES, 1), F32),
        ],
    )(x, W1, degp)


def _tc_combine(P, H, dis, inv, b, Wn):
    def body(p_ref, h_ref, dis_ref, inv_ref, b_ref, w_ref, hn_ref, hsn_ref):
        p = p_ref[...]
        dis = dis_ref[...]
        xv = jnp.maximum(dis * (p[0] + p[1]) + inv_ref[...] * h_ref[...]
                         + b_ref[...], 0.0)
        hn = jnp.dot(xv, w_ref[...], preferred_element_type=F32)
        hn_ref[...] = hn
        hsn_ref[...] = hn * dis

    return pl.pallas_call(
        body,
        grid=(GRID,),
        in_specs=[
            pl.BlockSpec((NC, R, D), lambda i: (0, i, 0)),
            pl.BlockSpec((R, D), lambda i: (i, 0)),
            pl.BlockSpec((R, 1), lambda i: (i, 0)),
            pl.BlockSpec((R, 1), lambda i: (i, 0)),
            pl.BlockSpec((1, D), lambda i: (0, 0)),
            pl.BlockSpec((D, D), lambda i: (0, 0)),
        ],
        out_specs=[
            pl.BlockSpec((R, D), lambda i: (i, 0)),
            pl.BlockSpec((R, D), lambda i: (i, 0)),
        ],
        out_shape=[
            jax.ShapeDtypeStruct((N_NODES, D), F32),
            jax.ShapeDtypeStruct((N_NODES, D), F32),
        ],
    )(P, H, dis, inv, b, Wn)


def _tc_combine_last(P, H, dis, inv, b):
    def body(p_ref, h_ref, dis_ref, inv_ref, b_ref, x_ref):
        p = p_ref[...]
        x_ref[...] = jnp.maximum(dis_ref[...] * (p[0] + p[1])
                                 + inv_ref[...] * h_ref[...] + b_ref[...], 0.0)

    return pl.pallas_call(
        body,
        grid=(GRID,),
        in_specs=[
            pl.BlockSpec((NC, R, D), lambda i: (0, i, 0)),
            pl.BlockSpec((R, D), lambda i: (i, 0)),
            pl.BlockSpec((R, 1), lambda i: (i, 0)),
            pl.BlockSpec((R, 1), lambda i: (i, 0)),
            pl.BlockSpec((1, D), lambda i: (0, 0)),
        ],
        out_specs=pl.BlockSpec((R, D), lambda i: (i, 0)),
        out_shape=jax.ShapeDtypeStruct((N_NODES, D), F32),
    )(P, H, dis, inv, b)


def _tc_final(Pp, cntp):
    def body(p_ref, c_ref, o_ref):
        p = p_ref[...]
        sums = p[0, :N_GRAPHS, :] + p[1, :N_GRAPHS, :]
        cn = c_ref[...]
        cnt = cn[0, :N_GRAPHS, 0:1] + cn[1, :N_GRAPHS, 0:1]
        o_ref[...] = sums / jnp.maximum(cnt, 1.0)

    return pl.pallas_call(
        body,
        in_specs=[
            pl.BlockSpec((NC, N_GRAPHS + 1, D), lambda: (0, 0, 0)),
            pl.BlockSpec((NC, N_GRAPHS + 1, D), lambda: (0, 0, 0)),
        ],
        out_specs=pl.BlockSpec((N_GRAPHS, D), lambda: (0, 0)),
        out_shape=jax.ShapeDtypeStruct((N_GRAPHS, D), F32),
    )(Pp, cntp)


def kernel(x, edge_index, batch, W1, b1, W2, b2, W3, b3):
    k_e = -(-N_EDGES // (NW * CHUNK))
    e_pad = k_e * NW * CHUNK - N_EDGES
    src = jnp.concatenate(
        [edge_index[0].astype(jnp.int32),
         jnp.arange(e_pad, dtype=jnp.int32) % N_NODES]
    ).reshape(NW, k_e, CHUNK)
    dst = jnp.concatenate(
        [edge_index[1].astype(jnp.int32),
         N_NODES + jnp.arange(e_pad, dtype=jnp.int32) % 128]
    ).reshape(NW, k_e, CHUNK)
    bat = batch.astype(jnp.int32)
    NALLOC = N_NODES + 128
    NALLOC_P = 128

    k_p = -(-N_NODES // (NW * CHUNK))
    n_pad = k_p * NW * CHUNK
    pool_src = jnp.concatenate(
        [jnp.arange(N_NODES, dtype=jnp.int32),
         jnp.arange(n_pad - N_NODES, dtype=jnp.int32)]).reshape(NW, k_p, CHUNK)
    pool_dst = jnp.concatenate(
        [bat, N_GRAPHS + jnp.arange(n_pad - N_NODES, dtype=jnp.int32)
         % (NALLOC_P - N_GRAPHS)]).reshape(NW, k_p, CHUNK)

    ones_e0 = jnp.zeros((CHUNK, D), F32).at[:, 0].set(1.0)
    z_nodes = jnp.zeros((N_NODES, D), F32)
    z_pool = jnp.zeros((N_GRAPHS + 1, D), F32)

    degp = _sc_hist(dst, ones_e0, z_nodes, N_NODES, NALLOC, k_e)
    cntp = _sc_hist(pool_dst, ones_e0, z_pool, N_GRAPHS + 1, NALLOC_P, k_p)

    b1r, b2r, b3r = (b.reshape(1, D) for b in (b1, b2, b3))

    H1, Hs1, dis, inv = _tc_prep(x, W1, degp)
    P1 = _sc_agg(Hs1, src, dst, z_nodes, N_NODES, NALLOC, k_e)
    H2, Hs2 = _tc_combine(P1, H1, dis, inv, b1r, W2)
    P2 = _sc_agg(Hs2, src, dst, z_nodes, N_NODES, NALLOC, k_e)
    H3, Hs3 = _tc_combine(P2, H2, dis, inv, b2r, W3)
    P3 = _sc_agg(Hs3, src, dst, z_nodes, N_NODES, NALLOC, k_e)
    X3 = _tc_combine_last(P3, H3, dis, inv, b3r)
    Pp = _sc_agg(X3, pool_src, pool_dst, z_pool, N_GRAPHS + 1,
                 NALLOC_P, k_p)
    return _tc_final(Pp, cntp)


# speedup vs baseline: 1.8835x; 1.0220x over previous
import functools

import jax
import jax.numpy as jnp
from jax import lax
from jax.experimental import pallas as pl
from jax.experimental.pallas import tpu as pltpu
from jax.experimental.pallas import tpu_sc as plsc

N_NODES = 10000
N_EDGES = 320000
D = 128
N_GRAPHS = 64

NC = 2
NS = 16
NW = NC * NS
CHUNK = 128

F32 = jnp.float32


def _span_copy(src, dst, s, nbins):
    rpt = (nbins // NS) & ~7
    if rpt == 0:
        @pl.when(s == 0)
        def _():
            pltpu.sync_copy(src.at[pl.ds(0, nbins)], dst.at[pl.ds(0, nbins)])
        return
    tail = nbins - NS * rpt
    base = pl.multiple_of(s * rpt, 8)
    pltpu.sync_copy(src.at[pl.ds(base, rpt)], dst.at[pl.ds(base, rpt)])
    if tail:
        @pl.when(s == NS - 1)
        def _():
            pltpu.sync_copy(src.at[pl.ds(NS * rpt, tail)],
                            dst.at[pl.ds(NS * rpt, tail)])

def _sc_hist(dst_rs, ones_e0, zeros, nbins, nalloc, k):
    mesh = plsc.VectorSubcoreMesh(core_axis_name="c", subcore_axis_name="s",
                                  num_cores=NC, num_subcores=NS)
    HBUF = 4

    @functools.partial(
        pl.kernel,
        out_type=jax.ShapeDtypeStruct((NC, nbins, D), F32),
        mesh=mesh,
        scratch_types=[
            pltpu.VMEM((k, CHUNK), jnp.int32),
            pltpu.VMEM((CHUNK, D), F32),
            pltpu.SemaphoreType.DMA((HBUF,)),
            pltpu.VMEM_SHARED((nalloc, D), F32),
        ],
    )
    def hist(dst_hbm, ones_hbm, zeros_hbm, out_hbm, idx_v, ones_v, semh, acc):
        c = lax.axis_index("c")
        s = lax.axis_index("s")
        wid = c * NS + s
        pltpu.sync_copy(dst_hbm.at[wid], idx_v)
        pltpu.sync_copy(ones_hbm, ones_v)
        _span_copy(zeros_hbm, acc, s, nbins)
        plsc.subcore_barrier()

        def wait_h(j, b):
            pltpu.make_async_copy(ones_v, acc.at[idx_v.at[j]],
                                  semh.at[b]).wait()

        def body(g, carry):
            for b in range(HBUF):
                pltpu.async_copy(ones_v, acc.at[idx_v.at[g * HBUF + b]],
                                 semh.at[b], add=True)
            for b in range(HBUF):
                wait_h(g * HBUF + b, b)
            return carry
        if k // HBUF:
            lax.fori_loop(0, k // HBUF, body, 0)
        for j in range(k - k % HBUF, k):
            pltpu.sync_copy(ones_v, acc.at[idx_v.at[j]], add=True)
        plsc.subcore_barrier()
        _span_copy(acc, out_hbm.at[c], s, nbins)

    return hist(dst_rs, ones_e0, zeros)


def _sc_agg(table, src_rs, dst_rs, zeros, nbins, nalloc, k):
    mesh = plsc.VectorSubcoreMesh(core_axis_name="c", subcore_axis_name="s",
                                  num_cores=NC, num_subcores=NS)

    NBUF = 2
    PH = 40
    phases = [(p0, min(PH, k - p0)) for p0 in range(0, k, PH)]
    kmax = max(kp for _, kp in phases)
    assert min(kp for _, kp in phases) >= NBUF

    @functools.partial(
        pl.kernel,
        out_type=jax.ShapeDtypeStruct((NC, nbins, D), F32),
        mesh=mesh,
        scratch_types=[
            pltpu.VMEM((kmax, CHUNK), jnp.int32),
            pltpu.VMEM((kmax, CHUNK), jnp.int32),
            pltpu.VMEM((NBUF, CHUNK, D), F32),
            pltpu.SemaphoreType.DMA((NBUF,)),
            pltpu.SemaphoreType.DMA((NBUF,)),
            pltpu.VMEM_SHARED((nalloc, D), F32),
        ],
    )
    def agg(table_hbm, src_hbm, dst_hbm, zeros_hbm, out_hbm,
            src_v, dst_v, rows_v, semg, sems, acc):
        c = lax.axis_index("c")
        s = lax.axis_index("s")
        wid = c * NS + s

        def wait_g(j, b):
            pltpu.make_async_copy(table_hbm.at[src_v.at[j]],
                                  rows_v.at[b], semg.at[b]).wait()

        def wait_s(j, b):
            pltpu.make_async_copy(rows_v.at[b],
                                  acc.at[dst_v.at[j]], sems.at[b]).wait()

        first = True
        for p0, kp in phases:
            pltpu.sync_copy(src_hbm.at[wid, pl.ds(p0, kp)],
                            src_v.at[pl.ds(0, kp)])
            pltpu.sync_copy(dst_hbm.at[wid, pl.ds(p0, kp)],
                            dst_v.at[pl.ds(0, kp)])
            for b in range(NBUF):
                pltpu.async_copy(table_hbm.at[src_v.at[b]], rows_v.at[b],
                                 semg.at[b])
            if first:
                _span_copy(zeros_hbm, acc, s, nbins)
                plsc.subcore_barrier()
                first = False

            n_main = (kp - NBUF) // NBUF
            base = n_main * NBUF

            def body(g, carry):
                for b in range(NBUF):
                    j = g * NBUF + b
                    wait_g(j, b)
                    pltpu.async_copy(rows_v.at[b], acc.at[dst_v.at[j]],
                                     sems.at[b], add=True)
                    wait_s(j, b)
                    pltpu.async_copy(table_hbm.at[src_v.at[j + NBUF]],
                                     rows_v.at[b], semg.at[b])
                return carry
            if n_main:
                lax.fori_loop(0, n_main, body, 0)

            for b in range(NBUF):
                wait_g(base + b, b)
                pltpu.async_copy(rows_v.at[b], acc.at[dst_v.at[base + b]],
                                 sems.at[b], add=True)
            for j in range(base + NBUF, kp):
                b = j % NBUF
                wait_s(j - NBUF, b)
                pltpu.sync_copy(table_hbm.at[src_v.at[j]], rows_v.at[b])
                pltpu.async_copy(rows_v.at[b], acc.at[dst_v.at[j]],
                                 sems.at[b], add=True)
            for b in range(NBUF):
                wait_s(kp - 1, b)
        plsc.subcore_barrier()
        _span_copy(acc, out_hbm.at[c], s, nbins)

    return agg(table, src_rs, dst_rs, zeros)


R = 400
GRID = N_NODES // R


def _tc_mm1(x, W1):
    def body(x_ref, w_ref, h_ref):
        h_ref[...] = jnp.dot(x_ref[...], w_ref[...], preferred_element_type=F32)

    return pl.pallas_call(
        body,
        grid=(GRID,),
        in_specs=[
            pl.BlockSpec((R, D), lambda i: (i, 0)),
            pl.BlockSpec((D, D), lambda i: (0, 0)),
        ],
        out_specs=pl.BlockSpec((R, D), lambda i: (i, 0)),
        out_shape=jax.ShapeDtypeStruct((N_NODES, D), F32),
    )(x, W1)


def _tc_scale(H1, degp):
    def body(h_ref, degp_ref, hs_ref, dis_ref, inv_ref):
        p = degp_ref[...]
        d = p[0, :, 0:1] + p[1, :, 0:1] + 1.0
        dis = lax.rsqrt(d)
        inv = 1.0 / d
        hs_ref[...] = h_ref[...] * dis
        dis_ref[...] = dis
        inv_ref[...] = inv

    return pl.pallas_call(
        body,
        grid=(GRID,),
        in_specs=[
            pl.BlockSpec((R, D), lambda i: (i, 0)),
            pl.BlockSpec((NC, R, D), lambda i: (0, i, 0)),
        ],
        out_specs=[
            pl.BlockSpec((R, D), lambda i: (i, 0)),
            pl.BlockSpec((R, 1), lambda i: (i, 0)),
            pl.BlockSpec((R, 1), lambda i: (i, 0)),
        ],
        out_shape=[
            jax.ShapeDtypeStruct((N_NODES, D), F32),
            jax.ShapeDtypeStruct((N_NODES, 1), F32),
            jax.ShapeDtypeStruct((N_NODES, 1), F32),
        ],
    )(H1, degp)


def _tc_combine(P, H, dis, inv, b, Wn):
    def body(p_ref, h_ref, dis_ref, inv_ref, b_ref, w_ref, hn_ref, hsn_ref):
        p = p_ref[...]
        dis = dis_ref[...]
        xv = jnp.maximum(dis * (p[0] + p[1]) + inv_ref[...] * h_ref[...]
                         + b_ref[...], 0.0)
        hn = jnp.dot(xv, w_ref[...], preferred_element_type=F32)
        hn_ref[...] = hn
        hsn_ref[...] = hn * dis

    return pl.pallas_call(
        body,
        grid=(GRID,),
        in_specs=[
            pl.BlockSpec((NC, R, D), lambda i: (0, i, 0)),
            pl.BlockSpec((R, D), lambda i: (i, 0)),
            pl.BlockSpec((R, 1), lambda i: (i, 0)),
            pl.BlockSpec((R, 1), lambda i: (i, 0)),
            pl.BlockSpec((1, D), lambda i: (0, 0)),
            pl.BlockSpec((D, D), lambda i: (0, 0)),
        ],
        out_specs=[
            pl.BlockSpec((R, D), lambda i: (i, 0)),
            pl.BlockSpec((R, D), lambda i: (i, 0)),
        ],
        out_shape=[
            jax.ShapeDtypeStruct((N_NODES, D), F32),
            jax.ShapeDtypeStruct((N_NODES, D), F32),
        ],
    )(P, H, dis, inv, b, Wn)


def _tc_combine3_pool(P, H, dis, inv, b, batch2d):
    def body(p_ref, h_ref, dis_ref, inv_ref, b_ref, bat_ref, o_ref,
             sum_acc, cnt_acc):
        i = pl.program_id(0)
        p = p_ref[...]
        xv = jnp.maximum(dis_ref[...] * (p[0] + p[1])
                         + inv_ref[...] * h_ref[...] + b_ref[...], 0.0)
        onehot = (lax.broadcasted_iota(jnp.int32, (R, N_GRAPHS), 1)
                  == bat_ref[...]).astype(F32)
        dn = (((0,), (0,)), ((), ()))
        contrib = lax.dot_general(onehot, xv, dn, preferred_element_type=F32)
        ccontrib = lax.dot_general(onehot, jnp.ones((R, 1), F32), dn,
                                   preferred_element_type=F32)

        @pl.when(i == 0)
        def _():
            sum_acc[...] = jnp.zeros_like(sum_acc)
            cnt_acc[...] = jnp.zeros_like(cnt_acc)

        sum_acc[...] += contrib
        cnt_acc[...] += ccontrib

        @pl.when(i == GRID - 1)
        def _():
            o_ref[...] = sum_acc[...] / jnp.maximum(cnt_acc[...], 1.0)

    return pl.pallas_call(
        body,
        grid=(GRID,),
        in_specs=[
            pl.BlockSpec((NC, R, D), lambda i: (0, i, 0)),
            pl.BlockSpec((R, D), lambda i: (i, 0)),
            pl.BlockSpec((R, 1), lambda i: (i, 0)),
            pl.BlockSpec((R, 1), lambda i: (i, 0)),
            pl.BlockSpec((1, D), lambda i: (0, 0)),
            pl.BlockSpec((R, 1), lambda i: (i, 0)),
        ],
        out_specs=pl.BlockSpec((N_GRAPHS, D), lambda i: (0, 0)),
        out_shape=jax.ShapeDtypeStruct((N_GRAPHS, D), F32),
        scratch_shapes=[
            pltpu.VMEM((N_GRAPHS, D), F32),
            pltpu.VMEM((N_GRAPHS, 1), F32),
        ],
    )(P, H, dis, inv, b, batch2d)


def kernel(x, edge_index, batch, W1, b1, W2, b2, W3, b3):
    k_e = -(-N_EDGES // (NW * CHUNK))
    e_pad = k_e * NW * CHUNK - N_EDGES
    src = jnp.concatenate(
        [edge_index[0].astype(jnp.int32),
         jnp.arange(e_pad, dtype=jnp.int32) % N_NODES]
    ).reshape(NW, k_e, CHUNK)
    dst = jnp.concatenate(
        [edge_index[1].astype(jnp.int32),
         N_NODES + jnp.arange(e_pad, dtype=jnp.int32) % 128]
    ).reshape(NW, k_e, CHUNK)
    batch2d = batch.astype(jnp.int32).reshape(N_NODES, 1)
    NALLOC = N_NODES + 128

    ones_e0 = jnp.zeros((CHUNK, D), F32).at[:, 0].set(1.0)
    z_nodes = jnp.zeros((N_NODES, D), F32)

    degp = _sc_hist(dst, ones_e0, z_nodes, N_NODES, NALLOC, k_e)
    H1 = _tc_mm1(x, W1)
    Hs1, dis, inv = _tc_scale(H1, degp)

    b1r, b2r, b3r = (b.reshape(1, D) for b in (b1, b2, b3))

    P1 = _sc_agg(Hs1, src, dst, z_nodes, N_NODES, NALLOC, k_e)
    H2, Hs2 = _tc_combine(P1, H1, dis, inv, b1r, W2)
    P2 = _sc_agg(Hs2, src, dst, z_nodes, N_NODES, NALLOC, k_e)
    H3, Hs3 = _tc_combine(P2, H2, dis, inv, b2r, W3)
    P3 = _sc_agg(Hs3, src, dst, z_nodes, N_NODES, NALLOC, k_e)
    return _tc_combine3_pool(P3, H3, dis, inv, b3r, batch2d)
